# baseline (speedup 1.0000x reference)
import math

import jax
import jax.numpy as jnp
import numpy as np
from jax import lax
from jax.experimental import pallas as pl
from jax.experimental.pallas import tpu as pltpu
from jax.experimental.pallas import tpu_sc as plsc

_NC = 2
_NS = 16
_NW = _NC * _NS
_LANES = 16


def _make_pe(seq_len: int, emb: int) -> np.ndarray:
    pe = np.zeros((seq_len, emb), dtype=np.float32)
    position = np.arange(0, seq_len, dtype=np.float32)[:, None]
    div_term = np.exp(
        np.arange(0, emb, 2, dtype=np.float32) * -(math.log(10000.0) / emb)
    )
    pe[:, 0::2] = np.sin(position * div_term)
    pe[:, 1::2] = np.cos(position * div_term)
    return pe


def _build(seq: int, batch: int, vocab: int, emb: int):
    assert batch % _NW == 0
    bw = batch // _NW
    vregs = emb // _LANES
    scale = float(math.sqrt(emb))
    mesh = plsc.VectorSubcoreMesh(core_axis_name="c", subcore_axis_name="s")

    @jax.jit
    def run(src_flat, table, pe_flat):
        def body(src_hbm, pe_hbm, table_hbm, out_hbm, idx_v, rows_v, pe_v, sem):
            wid = lax.axis_index("s") * _NC + lax.axis_index("c")
            pltpu.sync_copy(pe_hbm, pe_v)

            def step(s, carry):
                base = s * batch + wid * bw
                pltpu.sync_copy(src_hbm.at[pl.ds(base, bw)], idx_v)
                pltpu.async_copy(table_hbm.at[idx_v], rows_v, sem).wait()
                pe_vecs = [
                    pe_v[pl.ds(s * emb + j * _LANES, _LANES)]
                    for j in range(vregs)
                ]

                def row(i, c2):
                    for j in range(vregs):
                        sl = pl.ds(j * _LANES, _LANES)
                        rows_v[i, sl] = rows_v[i, sl] * scale + pe_vecs[j]
                    return c2

                lax.fori_loop(0, bw, row, 0, unroll=2)
                pltpu.sync_copy(rows_v, out_hbm.at[pl.ds(base, bw)])
                return carry

            lax.fori_loop(0, seq, step, 0)

        return pl.kernel(
            body,
            out_type=jax.ShapeDtypeStruct((seq * batch, emb), jnp.float32),
            mesh=mesh,
            scratch_types=[
                pltpu.VMEM((bw,), jnp.int32),
                pltpu.VMEM((bw, emb), jnp.float32),
                pltpu.VMEM((seq * emb,), jnp.float32),
                pltpu.SemaphoreType.DMA,
            ],
            compiler_params=pltpu.CompilerParams(use_tc_tiling_on_sc=False),
        )(src_flat, pe_flat, table)

    return run


def kernel(src, table, step=0):
    seq, batch = src.shape
    vocab, emb = table.shape
    run = _build(seq, batch, vocab, emb)
    pe_flat = jnp.asarray(_make_pe(seq, emb).reshape(-1))
    src_flat = src.astype(jnp.int32).reshape(-1)
    out = run(src_flat, table, pe_flat)
    return out.reshape(seq, batch, emb)


# speedup vs baseline: 1.2028x; 1.2028x over previous
import math

import jax
import jax.numpy as jnp
import numpy as np
from jax import lax
from jax.experimental import pallas as pl
from jax.experimental.pallas import tpu as pltpu
from jax.experimental.pallas import tpu_sc as plsc

_NC = 2
_NS = 16
_NW = _NC * _NS
_LANES = 16


def _make_pe(seq_len: int, emb: int) -> np.ndarray:
    pe = np.zeros((seq_len, emb), dtype=np.float32)
    position = np.arange(0, seq_len, dtype=np.float32)[:, None]
    div_term = np.exp(
        np.arange(0, emb, 2, dtype=np.float32) * -(math.log(10000.0) / emb)
    )
    pe[:, 0::2] = np.sin(position * div_term)
    pe[:, 1::2] = np.cos(position * div_term)
    return pe


def _build(seq: int, batch: int, vocab: int, emb: int):
    assert batch % _NW == 0 and seq % 2 == 0
    bw = batch // _NW
    vregs = emb // _LANES
    scale = float(math.sqrt(emb))
    mesh = plsc.VectorSubcoreMesh(core_axis_name="c", subcore_axis_name="s")

    @jax.jit
    def run(src, table, pe_flat):
        def body(src_hbm, pe_hbm, table_hbm, out_hbm,
                 idx_all, pe_v, rows0, rows1, g0, g1, st0, st1):
            wid = lax.axis_index("s") * _NC + lax.axis_index("c")
            boff = wid * bw
            pltpu.sync_copy(src_hbm.at[:, pl.ds(boff, bw)], idx_all)
            pltpu.sync_copy(pe_hbm, pe_v)

            rows = (rows0, rows1)
            gsem = (g0, g1)
            ssem = (st0, st1)

            def gather_start(s, b):
                pltpu.async_copy(table_hbm.at[idx_all.at[s]], rows[b], gsem[b])

            def gather_wait(b):
                pltpu.make_async_copy(
                    table_hbm.at[idx_all.at[0]], rows[b], gsem[b]
                ).wait()

            def store_start(s, b):
                pltpu.async_copy(
                    rows[b], out_hbm.at[s, pl.ds(boff, bw)], ssem[b]
                )

            def store_wait(b):
                pltpu.make_async_copy(
                    rows[b], out_hbm.at[0, pl.ds(boff, bw)], ssem[b]
                ).wait()

            def compute(s, b):
                pe_vecs = [
                    pe_v[pl.ds(s * emb + j * _LANES, _LANES)]
                    for j in range(vregs)
                ]
                buf = rows[b]

                @pl.loop(0, bw, unroll=4)
                def _row(i):
                    for j in range(vregs):
                        sl = pl.ds(j * _LANES, _LANES)
                        buf[i, sl] = buf[i, sl] * scale + pe_vecs[j]

            gather_start(0, 0)

            @pl.loop(0, seq, step=2)
            def _iter(g):
                for b in range(2):
                    s = g + b
                    nxt = 1 - b

                    @pl.when(s + 1 < seq)
                    def _prefetch():
                        @pl.when(s >= 1)
                        def _drain():
                            store_wait(nxt)
                        gather_start(s + 1, nxt)

                    gather_wait(b)
                    compute(s, b)
                    store_start(s, b)

            store_wait(0)
            store_wait(1)

        return pl.kernel(
            body,
            out_type=jax.ShapeDtypeStruct((seq, batch, emb), jnp.float32),
            mesh=mesh,
            scratch_types=[
                pltpu.VMEM((seq, bw), jnp.int32),
                pltpu.VMEM((seq * emb,), jnp.float32),
                pltpu.VMEM((bw, emb), jnp.float32),
                pltpu.VMEM((bw, emb), jnp.float32),
                pltpu.SemaphoreType.DMA,
                pltpu.SemaphoreType.DMA,
                pltpu.SemaphoreType.DMA,
                pltpu.SemaphoreType.DMA,
            ],
            compiler_params=pltpu.CompilerParams(use_tc_tiling_on_sc=False),
        )(src, pe_flat, table)

    return run


def kernel(src, table, step=0):
    seq, batch = src.shape
    vocab, emb = table.shape
    run = _build(seq, batch, vocab, emb)
    pe_flat = jnp.asarray(_make_pe(seq, emb).reshape(-1))
    return run(src.astype(jnp.int32), table, pe_flat)
